# baseline (speedup 1.0000x reference)
import functools

import jax
import jax.numpy as jnp
from jax import lax
from jax.experimental import pallas as pl
from jax.experimental.pallas import tpu as pltpu
from jax.experimental.pallas import tpu_sc as plsc

N = 10000
E = 320000
D = 128
H = 128
G = 64

NC = 2
NS = 16
EPT = E // NS
EPTP = 20480
CH = 80
NCH = EPTP // CH
HCH = NCH // 2
NHALF = 5120
JUNK = NHALF
NACC = NHALF + 64
RPT = NHALF // NS

BR = 1000
GR = N // BR


def _seg_sum_body(x_hbm, src_hbm, dst_hbm, out_hbm,
                  src_v, dst_v, rows_0, rows_1, rows_2, acc,
                  g0, g1, g2, s0, s1, s2):
    cid = lax.axis_index("c")
    sid = lax.axis_index("s")

    def zrow(r, carry):
        for cc in range(H // 16):
            rows_0[r, pl.ds(cc * 16, 16)] = jnp.zeros((16,), jnp.float32)
        return carry
    lax.fori_loop(0, CH, zrow, 0)
    for k in range(RPT // CH):
        pltpu.sync_copy(rows_0, acc.at[pl.ds(sid * RPT + k * CH, CH)])

    @pl.when(sid == 0)
    def _():
        pltpu.sync_copy(rows_0.at[pl.ds(0, 64)], acc.at[pl.ds(NHALF, 64)])
    plsc.subcore_barrier()

    bufs = (rows_0, rows_1, rows_2)
    gsems = (g0, g1, g2)
    ssems = (s0, s1, s2)
    lo = cid * NHALF

    def gstart(ci, b):
        pltpu.async_copy(x_hbm.at[src_v.at[ci]], bufs[b], gsems[b])

    def gdrain(b):
        pltpu.make_async_copy(x_hbm.at[pl.ds(0, CH)], bufs[b],
                              gsems[b]).wait()

    def sstart(ci, b):
        pltpu.async_copy(bufs[b], acc.at[dst_v.at[ci]], ssems[b], add=True)

    def sdrain(b):
        pltpu.make_async_copy(x_hbm.at[pl.ds(0, CH)], bufs[b],
                              ssems[b]).wait()

    def half(ph):
        pltpu.sync_copy(src_hbm.at[sid, pl.ds(ph * HCH, HCH)], src_v)
        pltpu.sync_copy(dst_hbm.at[sid, pl.ds(ph * HCH, HCH)], dst_v)

        def remap(ci, carry):
            for cc in range(CH // 16):
                v = dst_v[ci, pl.ds(cc * 16, 16)]
                vl = v - lo
                ok = (vl >= 0) & (vl < NHALF)
                dst_v[ci, pl.ds(cc * 16, 16)] = jnp.where(ok, vl, JUNK)
            return carry
        lax.fori_loop(0, HCH, remap, 0)

        gstart(0, 0)
        gstart(1, 1)

        def triple(ti, carry):
            c0 = ti * 3
            for jj in range(3):
                c = c0 + jj
                j = jj
                jn = (jj + 2) % 3
                gdrain(j)
                sstart(c, j)
                if jj == 0:
                    @pl.when(ti > 0)
                    def _():
                        sdrain(jn)
                else:
                    sdrain(jn)

                @pl.when(c + 2 < HCH)
                def _():
                    gstart(c + 2, jn)
            return carry
        lax.fori_loop(0, HCH // 3, triple, 0)
        for c in (126, 127):
            j = c % 3
            gdrain(j)
            sstart(c, j)
            sdrain((j + 2) % 3)
        sdrain((HCH - 1) % 3)

    half(0)
    half(1)

    plsc.subcore_barrier()
    pltpu.sync_copy(acc.at[pl.ds(sid * RPT, RPT)],
                    out_hbm.at[cid, pl.ds(sid * RPT, RPT)])


@functools.cache
def _seg_sum_kernel():
    return pl.kernel(
        _seg_sum_body,
        mesh=plsc.VectorSubcoreMesh(core_axis_name="c", subcore_axis_name="s"),
        out_type=jax.ShapeDtypeStruct((NC, NHALF, H), jnp.float32),
        scratch_types=[
            pltpu.VMEM((HCH, CH), jnp.int32),
            pltpu.VMEM((HCH, CH), jnp.int32),
            pltpu.VMEM((CH, H), jnp.float32),
            pltpu.VMEM((CH, H), jnp.float32),
            pltpu.VMEM((CH, H), jnp.float32),
            pltpu.VMEM_SHARED((NACC, H), jnp.float32),
            pltpu.SemaphoreType.DMA,
            pltpu.SemaphoreType.DMA,
            pltpu.SemaphoreType.DMA,
            pltpu.SemaphoreType.DMA,
            pltpu.SemaphoreType.DMA,
            pltpu.SemaphoreType.DMA,
        ],
    )


def _layer_body(p_ref, x_ref, b_ref, wr_ref, br_ref, wo_ref,
                x_out_ref, h_ref):
    i = pl.program_id(0)
    cdims = (((1,), (1,)), ((), ()))
    t = (lax.dot_general(p_ref[...], wr_ref[...], cdims,
                         preferred_element_type=jnp.float32)
         + br_ref[...]
         + lax.dot_general(x_ref[...], wo_ref[...], cdims,
                           preferred_element_type=jnp.float32))
    m = jnp.mean(t, axis=1, keepdims=True)
    v = jnp.mean((t - m) ** 2, axis=1, keepdims=True)
    xn = jnp.maximum((t - m) * lax.rsqrt(v + 1e-5), 0.0)
    x_out_ref[...] = xn

    b = b_ref[0]

    @pl.when(i == 0)
    def _():
        h_ref[...] = jnp.full((G, H), -jnp.inf, jnp.float32)

    glo = jnp.min(b)
    ghi = jnp.max(b)

    def gbody(g, carry):
        sel = jnp.where(b == g, xn, -jnp.inf)
        mx = jnp.max(sel, axis=0, keepdims=True)
        h_ref[pl.ds(g, 1), :] = jnp.maximum(h_ref[pl.ds(g, 1), :], mx)
        return carry
    lax.fori_loop(glo, ghi + 1, gbody, 0)


def _layer_in_specs():
    return [
        pl.BlockSpec((BR, H), lambda i: (i, 0)),
        pl.BlockSpec((BR, H), lambda i: (i, 0)),
        pl.BlockSpec((1, BR, 1), lambda i: (i, 0, 0)),
        pl.BlockSpec((H, H), lambda i: (0, 0)),
        pl.BlockSpec((1, H), lambda i: (0, 0)),
        pl.BlockSpec((H, H), lambda i: (0, 0)),
    ]


_layer_tc = pl.pallas_call(
    _layer_body,
    grid=(GR,),
    in_specs=_layer_in_specs(),
    out_specs=[pl.BlockSpec((BR, H), lambda i: (i, 0)),
               pl.BlockSpec((G, H), lambda i: (0, 0))],
    out_shape=[jax.ShapeDtypeStruct((N, H), jnp.float32),
               jax.ShapeDtypeStruct((G, H), jnp.float32)],
)


def _mlp_body(h1_ref, h2_ref, h3_ref, wl1_ref, bl1_ref, wl2_ref, bl2_ref,
              out_ref):
    h = jnp.concatenate([h1_ref[...], h2_ref[...], h3_ref[...]], axis=1)
    cdims = (((1,), (1,)), ((), ()))
    t = lax.dot_general(h, wl1_ref[...], cdims,
                        preferred_element_type=jnp.float32) + bl1_ref[...]
    t = jnp.maximum(t, 0.0)
    o = lax.dot_general(t, wl2_ref[...], cdims,
                        preferred_element_type=jnp.float32) + bl2_ref[...]
    nrm = jnp.sqrt(jnp.sum(o * o, axis=1, keepdims=True))
    out_ref[...] = o / jnp.maximum(nrm, 1e-12)


_mlp = pl.pallas_call(
    _mlp_body,
    out_shape=jax.ShapeDtypeStruct((G, H // 2), jnp.float32),
)


def kernel(x, edge_index, batch, Wr1, br1, Wo1, Wr2, br2, Wo2, Wr3, br3, Wo3,
           Wl1, bl1, Wl2, bl2):
    pad = EPTP - EPT
    src3 = jnp.pad(edge_index[0].astype(jnp.int32).reshape(NS, EPT),
                   ((0, 0), (0, pad))).reshape(NS, NCH, CH)
    dst3 = jnp.pad(edge_index[1].astype(jnp.int32).reshape(NS, EPT),
                   ((0, 0), (0, pad)),
                   constant_values=-1).reshape(NS, NCH, CH)
    batch_r = batch.astype(jnp.int32).reshape(GR, BR, 1)

    seg = _seg_sum_kernel()

    Wr = jnp.stack([Wr1, Wr2, Wr3])
    brs = jnp.stack([br1.reshape(1, H), br2.reshape(1, H), br3.reshape(1, H)])
    Wo = jnp.stack([Wo1, Wo2, Wo3])

    def step(xin, w):
        wr, br_l, wo = w
        p = seg(xin, src3, dst3).reshape(NC * NHALF, H)
        xn, h = _layer_tc(p, xin, batch_r, wr, br_l, wo)
        return xn, h

    _, hs = lax.scan(step, x, (Wr, brs, Wo))

    return _mlp(hs[0], hs[1], hs[2], Wl1, bl1.reshape(1, 2 * H), Wl2,
                bl2.reshape(1, H // 2))


# speedup vs baseline: 2.9162x; 2.9162x over previous
import functools

import jax
import jax.numpy as jnp
from jax import lax
from jax.experimental import pallas as pl
from jax.experimental.pallas import tpu as pltpu
from jax.experimental.pallas import tpu_sc as plsc

N = 10000
E = 320000
D = 128
H = 128
G = 64

NC = 2
NS = 16
EPT = E // NS
CH = 80
NCH = EPT // CH
NHALF = 5120
JUNK = NHALF
NACC = NHALF + 64
RPT = NHALF // NS

BR = 1000
GR = N // BR


def _seg_sum_body(x_hbm, src_hbm, dst_hbm, out_hbm,
                  src_v, dst_v, rows_v, rows_w, acc, g0, g1, s0, s1):
    cid = lax.axis_index("c")
    sid = lax.axis_index("s")

    def zrow(r, carry):
        for cc in range(H // 16):
            rows_v[r, pl.ds(cc * 16, 16)] = jnp.zeros((16,), jnp.float32)
        return carry
    lax.fori_loop(0, CH, zrow, 0)
    for k in range(RPT // CH):
        pltpu.sync_copy(rows_v, acc.at[pl.ds(sid * RPT + k * CH, CH)])

    @pl.when(sid == 0)
    def _():
        pltpu.sync_copy(rows_v.at[pl.ds(0, 64)], acc.at[pl.ds(NHALF, 64)])

    pltpu.sync_copy(src_hbm.at[sid], src_v)
    pltpu.sync_copy(dst_hbm.at[sid], dst_v)
    lo = cid * NHALF

    def remap(ci, carry):
        for cc in range(CH // 16):
            v = dst_v[ci, pl.ds(cc * 16, 16)]
            vl = v - lo
            ok = (vl >= 0) & (vl < NHALF)
            dst_v[ci, pl.ds(cc * 16, 16)] = jnp.where(ok, vl, JUNK)
        return carry
    lax.fori_loop(0, NCH, remap, 0)
    plsc.subcore_barrier()

    bufs = (rows_v, rows_w)
    gsems = (g0, g1)
    ssems = (s0, s1)

    def gstart(ci, b):
        pltpu.async_copy(x_hbm.at[src_v.at[ci]], bufs[b], gsems[b])

    def gdrain(b):
        pltpu.make_async_copy(x_hbm.at[pl.ds(0, CH)], bufs[b],
                              gsems[b]).wait()

    def sstart(ci, b):
        pltpu.async_copy(bufs[b], acc.at[dst_v.at[ci]], ssems[b], add=True)

    def sdrain(b):
        pltpu.make_async_copy(x_hbm.at[pl.ds(0, CH)], bufs[b],
                              ssems[b]).wait()

    gstart(0, 0)
    gstart(1, 1)

    def pair(pi, carry):
        c = pi * 2
        gdrain(0)
        sstart(c, 0)

        @pl.when(c + 2 < NCH)
        def _():
            sdrain(0)
            gstart(c + 2, 0)
        gdrain(1)
        sstart(c + 1, 1)

        @pl.when(c + 3 < NCH)
        def _():
            sdrain(1)
            gstart(c + 3, 1)
        return carry
    lax.fori_loop(0, NCH // 2, pair, 0)
    sdrain(0)
    sdrain(1)

    plsc.subcore_barrier()
    pltpu.sync_copy(acc.at[pl.ds(sid * RPT, RPT)],
                    out_hbm.at[cid, pl.ds(sid * RPT, RPT)])


@functools.cache
def _seg_sum_kernel():
    return pl.kernel(
        _seg_sum_body,
        mesh=plsc.VectorSubcoreMesh(core_axis_name="c", subcore_axis_name="s"),
        out_type=jax.ShapeDtypeStruct((NC, NHALF, H), jnp.float32),
        scratch_types=[
            pltpu.VMEM((NCH, CH), jnp.int32),
            pltpu.VMEM((NCH, CH), jnp.int32),
            pltpu.VMEM((CH, H), jnp.float32),
            pltpu.VMEM((CH, H), jnp.float32),
            pltpu.VMEM_SHARED((NACC, H), jnp.float32),
            pltpu.SemaphoreType.DMA,
            pltpu.SemaphoreType.DMA,
            pltpu.SemaphoreType.DMA,
            pltpu.SemaphoreType.DMA,
        ],
    )


def _layer_body(p_ref, x_ref, b_ref, wr_ref, br_ref, wo_ref,
                x_out_ref, h_ref):
    i = pl.program_id(0)
    cdims = (((1,), (1,)), ((), ()))
    t = (lax.dot_general(p_ref[...], wr_ref[...], cdims,
                         preferred_element_type=jnp.float32)
         + br_ref[...]
         + lax.dot_general(x_ref[...], wo_ref[...], cdims,
                           preferred_element_type=jnp.float32))
    m = jnp.mean(t, axis=1, keepdims=True)
    v = jnp.mean((t - m) ** 2, axis=1, keepdims=True)
    xn = jnp.maximum((t - m) * lax.rsqrt(v + 1e-5), 0.0)
    x_out_ref[...] = xn

    b = b_ref[0]

    @pl.when(i == 0)
    def _():
        h_ref[...] = jnp.full((G, H), -jnp.inf, jnp.float32)

    glo = jnp.min(b)
    ghi = jnp.max(b)

    def gbody(g, carry):
        sel = jnp.where(b == g, xn, -jnp.inf)
        mx = jnp.max(sel, axis=0, keepdims=True)
        h_ref[pl.ds(g, 1), :] = jnp.maximum(h_ref[pl.ds(g, 1), :], mx)
        return carry
    lax.fori_loop(glo, ghi + 1, gbody, 0)


def _layer_in_specs():
    return [
        pl.BlockSpec((BR, H), lambda i: (i, 0)),
        pl.BlockSpec((BR, H), lambda i: (i, 0)),
        pl.BlockSpec((1, BR, 1), lambda i: (i, 0, 0)),
        pl.BlockSpec((H, H), lambda i: (0, 0)),
        pl.BlockSpec((1, H), lambda i: (0, 0)),
        pl.BlockSpec((H, H), lambda i: (0, 0)),
    ]


_layer_tc = pl.pallas_call(
    _layer_body,
    grid=(GR,),
    in_specs=_layer_in_specs(),
    out_specs=[pl.BlockSpec((BR, H), lambda i: (i, 0)),
               pl.BlockSpec((G, H), lambda i: (0, 0))],
    out_shape=[jax.ShapeDtypeStruct((N, H), jnp.float32),
               jax.ShapeDtypeStruct((G, H), jnp.float32)],
)


def _mlp_body(h1_ref, h2_ref, h3_ref, wl1_ref, bl1_ref, wl2_ref, bl2_ref,
              out_ref):
    h = jnp.concatenate([h1_ref[...], h2_ref[...], h3_ref[...]], axis=1)
    cdims = (((1,), (1,)), ((), ()))
    t = lax.dot_general(h, wl1_ref[...], cdims,
                        preferred_element_type=jnp.float32) + bl1_ref[...]
    t = jnp.maximum(t, 0.0)
    o = lax.dot_general(t, wl2_ref[...], cdims,
                        preferred_element_type=jnp.float32) + bl2_ref[...]
    nrm = jnp.sqrt(jnp.sum(o * o, axis=1, keepdims=True))
    out_ref[...] = o / jnp.maximum(nrm, 1e-12)


_mlp = pl.pallas_call(
    _mlp_body,
    out_shape=jax.ShapeDtypeStruct((G, H // 2), jnp.float32),
)


def kernel(x, edge_index, batch, Wr1, br1, Wo1, Wr2, br2, Wo2, Wr3, br3, Wo3,
           Wl1, bl1, Wl2, bl2):
    src3 = edge_index[0].astype(jnp.int32).reshape(NS, NCH, CH)
    dst3 = edge_index[1].astype(jnp.int32).reshape(NS, NCH, CH)
    batch_r = batch.astype(jnp.int32).reshape(GR, BR, 1)

    seg = _seg_sum_kernel()

    Wr = jnp.stack([Wr1, Wr2, Wr3])
    brs = jnp.stack([br1.reshape(1, H), br2.reshape(1, H), br3.reshape(1, H)])
    Wo = jnp.stack([Wo1, Wo2, Wo3])

    def step(xin, w):
        wr, br_l, wo = w
        p = seg(xin, src3, dst3).reshape(NC * NHALF, H)
        xn, h = _layer_tc(p, xin, batch_r, wr, br_l, wo)
        return xn, h

    _, hs = lax.scan(step, x, (Wr, brs, Wo))

    return _mlp(hs[0], hs[1], hs[2], Wl1, bl1.reshape(1, 2 * H), Wl2,
                bl2.reshape(1, H // 2))
